# baseline (speedup 1.0000x reference)
import functools

import jax
import jax.numpy as jnp
from jax import lax
from jax.experimental import pallas as pl
from jax.experimental.pallas import tpu as pltpu
from jax.experimental.pallas import tpu_sc as plsc

B = 16
G = 64
S = 16
D = 128
N_GENES = 20000

NC = 2
NS = 16
NW = NC * NS
LANES = 16

GROUPS_PER_W = (B * G) // NW
ROWS_PER_W = GROUPS_PER_W * S
HALVES = G // GROUPS_PER_W


NCHUNK = 4
CROWS = ROWS_PER_W // NCHUNK
CGROUPS = GROUPS_PER_W // NCHUNK


def _sc_body(x_hbm, out_hbm, in_v, out_v, sem0, sem1):
    wid = lax.axis_index("s") * NC + lax.axis_index("c")
    b = wid // HALVES
    half = wid % HALVES
    in_base = b * N_GENES + half * ROWS_PER_W
    out_base = wid * GROUPS_PER_W
    sems = (sem0, sem1)

    def start(c):
        return pltpu.async_copy(
            x_hbm.at[pl.ds(in_base + c * CROWS, CROWS), :],
            in_v.at[c % 2], sems[c % 2])

    copies = [start(0)]
    for c in range(NCHUNK):
        if c + 1 < NCHUNK:
            copies.append(start(c + 1))
        copies[c].wait()
        buf = c % 2
        gbase = c * CGROUPS

        def gbody(g, carry):
            row0 = g * S
            for dc in range(D // LANES):
                sl = pl.ds(dc * LANES, LANES)
                acc = in_v[buf, row0, sl]
                for s in range(1, S):
                    acc = acc + in_v[buf, row0 + s, sl]
                out_v[gbase + g, sl] = acc * (1.0 / S)
            return carry

        lax.fori_loop(0, CGROUPS, gbody, 0)

    pltpu.sync_copy(out_v, out_hbm.at[pl.ds(out_base, GROUPS_PER_W), :])


_sc_kernel = functools.partial(
    pl.kernel,
    out_type=jax.ShapeDtypeStruct((B * G, D), jnp.float32),
    mesh=plsc.VectorSubcoreMesh(core_axis_name="c", subcore_axis_name="s"),
    scratch_types=[
        pltpu.VMEM((2, CROWS, D), jnp.float32),
        pltpu.VMEM((GROUPS_PER_W, D), jnp.float32),
        pltpu.SemaphoreType.DMA,
        pltpu.SemaphoreType.DMA,
    ],
)(_sc_body)


@jax.jit
def kernel(gene_output):
    flat = gene_output.reshape(B * N_GENES, D)
    out = _sc_kernel(flat)
    return out.reshape(B, G, D)


# speedup vs baseline: 1.1826x; 1.1826x over previous
import functools

import jax
import jax.numpy as jnp
from jax import lax
from jax.experimental import pallas as pl
from jax.experimental.pallas import tpu as pltpu
from jax.experimental.pallas import tpu_sc as plsc

B = 16
G = 64
S = 16
D = 128
N_GENES = 20000

NC = 2
NS = 16
NW = NC * NS
LANES = 16

GROUPS_PER_W = (B * G) // NW
ROWS_PER_W = GROUPS_PER_W * S
HALVES = G // GROUPS_PER_W


def _sc_body(x_hbm, out_hbm, in_v, out_v):
    wid = lax.axis_index("s") * NC + lax.axis_index("c")
    b = wid // HALVES
    half = wid % HALVES
    in_base = b * N_GENES + half * ROWS_PER_W
    out_base = wid * GROUPS_PER_W

    pltpu.sync_copy(x_hbm.at[pl.ds(in_base, ROWS_PER_W), :], in_v)

    @plsc.parallel_loop(0, GROUPS_PER_W, unroll=2)
    def gbody(g):
        row0 = g * S
        for dc in range(D // LANES):
            sl = pl.ds(dc * LANES, LANES)
            vals = [in_v[row0 + s, sl] for s in range(S)]
            while len(vals) > 1:
                vals = [vals[i] + vals[i + 1] for i in range(0, len(vals), 2)]
            out_v[g, sl] = vals[0] * (1.0 / S)

    pltpu.sync_copy(out_v, out_hbm.at[pl.ds(out_base, GROUPS_PER_W), :])


_sc_kernel = functools.partial(
    pl.kernel,
    out_type=jax.ShapeDtypeStruct((B * G, D), jnp.float32),
    mesh=plsc.VectorSubcoreMesh(core_axis_name="c", subcore_axis_name="s"),
    scratch_types=[
        pltpu.VMEM((ROWS_PER_W, D), jnp.float32),
        pltpu.VMEM((GROUPS_PER_W, D), jnp.float32),
    ],
)(_sc_body)


@jax.jit
def kernel(gene_output):
    flat = gene_output.reshape(B * N_GENES, D)
    out = _sc_kernel(flat)
    return out.reshape(B, G, D)
